# Initial kernel scaffold; baseline (speedup 1.0000x reference)
import jax
import jax.numpy as jnp
from jax.experimental import pallas as pl


def kernel(inputs, input_paddings, conv1_w, conv1_b, conv2_w, conv2_b, lin_w, lin_b):
    raise NotImplementedError("write your pallas kernel here")


# speedup vs baseline: 1.0749x; 1.0749x over previous
import jax
import jax.numpy as jnp
from jax.experimental import pallas as pl
from jax.experimental.pallas import tpu as pltpu


_HALO = 8


def _fused_kernel(p_ref, ph_ref, pd1_ref, pd1h_ref, pd2_ref,
                  w1_ref, b1_ref, w2_ref, b2_ref, wl_ref, bl_ref,
                  out_ref):
    tile2 = out_ref.shape[1]
    fo1 = p_ref.shape[2]
    k1 = p_ref.shape[3]
    co = w2_ref.shape[3]
    fo2 = fo1 // 2

    def conv1(p, pd, n):
        y = jnp.dot(p.reshape(n * fo1, k1), w1_ref[...],
                    preferred_element_type=jnp.float32)
        y = jnp.maximum(y + b1_ref[...], 0.0)
        y = y.reshape(n, fo1, co) * (1.0 - pd).reshape(n, 1, 1)
        return y.astype(jnp.bfloat16)

    c1m = conv1(p_ref[0], pd1_ref[0], 2 * tile2)
    c1h = conv1(ph_ref[0], pd1h_ref[0], ph_ref.shape[1])

    c1p = c1m.reshape(tile2, 2, fo1, co)
    r_even = c1p[:, 0]
    r_odd = c1p[:, 1]
    r_next = jnp.concatenate([r_even[1:], c1h[:1]], axis=0)

    acc = jnp.zeros((tile2 * fo2, co), jnp.float32)
    for kh, rset in enumerate((r_even, r_odd, r_next)):
        rp = rset.reshape(tile2, fo2, 2, co)
        e = rp[:, :, 0]
        o = rp[:, :, 1]
        e2 = jnp.concatenate([e[:, 1:], jnp.zeros_like(e[:, :1])], axis=1)
        for kw, tap in enumerate((e, o, e2)):
            acc = acc + jnp.dot(tap.reshape(tile2 * fo2, co), w2_ref[kh, kw],
                                preferred_element_type=jnp.float32)

    y = jnp.maximum(acc + b2_ref[...], 0.0)
    y = y.reshape(tile2, fo2, co) * (1.0 - pd2_ref[0]).reshape(tile2, 1, 1)
    a = y.astype(jnp.bfloat16).reshape(tile2, fo2 * co)
    out = jnp.dot(a, wl_ref[...], preferred_element_type=jnp.float32)
    out_ref[0] = out + bl_ref[...]


def _subsample_pad(p):
    t = p.shape[1]
    if t % 2:
        p = jnp.pad(p, ((0, 0), (0, 1)), constant_values=1.0)
    return p[:, ::2]


def kernel(inputs, input_paddings, conv1_w, conv1_b, conv2_w, conv2_b,
           lin_w, lin_b, *, tile2=32):
    x = inputs.astype(jnp.float32)
    B, T, F = x.shape
    C = conv1_w.shape[-1]
    Co = conv2_w.shape[-1]
    D = lin_w.shape[-1]
    To1, Fo1 = (T + 1) // 2, (F + 1) // 2
    To2, Fo2 = (To1 + 1) // 2, (Fo1 + 1) // 2

    n_t = pl.cdiv(To2, tile2)
    Tp2 = n_t * tile2
    S1 = 2 * Tp2 + _HALO

    out_pad1 = _subsample_pad(input_paddings)
    out_pad2 = _subsample_pad(out_pad1)
    pads1 = jnp.pad(out_pad1, ((0, 0), (0, S1 - To1)),
                    constant_values=1.0)[..., None]
    pads2 = jnp.pad(out_pad2, ((0, 0), (0, Tp2 - To2)),
                    constant_values=1.0)[..., None]

    xp = jnp.pad(x, ((0, 0), (0, 2 * S1 + 2 - T), (0, 2 * Fo1 + 2 - F)))
    slabs = [xp[:, dh:dh + 2 * S1:2, dw:dw + 2 * Fo1:2]
             for dh in range(3) for dw in range(3)]
    patches = jnp.stack(slabs, axis=-1)
    patches = jnp.pad(patches, ((0, 0), (0, 0), (0, 0), (0, 7))
                      ).astype(jnp.bfloat16)

    w1p = jnp.zeros((16, C), jnp.float32).at[:9].set(conv1_w.reshape(9, C))
    w1p = w1p.astype(jnp.bfloat16)
    b1r = conv1_b.reshape(1, C).astype(jnp.float32)
    w2r = conv2_w.astype(jnp.bfloat16)
    b2r = conv2_b.reshape(1, Co).astype(jnp.float32)
    wlr = lin_w.astype(jnp.bfloat16)
    blr = lin_b.reshape(1, D).astype(jnp.float32)

    rows = 2 * tile2
    halo_idx = rows // _HALO

    out = pl.pallas_call(
        _fused_kernel,
        out_shape=jax.ShapeDtypeStruct((B, Tp2, D), jnp.float32),
        grid=(B, n_t),
        in_specs=[
            pl.BlockSpec((1, rows, Fo1, 16), lambda b, t: (b, t, 0, 0)),
            pl.BlockSpec((1, _HALO, Fo1, 16),
                         lambda b, t: (b, (t + 1) * halo_idx, 0, 0)),
            pl.BlockSpec((1, rows, 1), lambda b, t: (b, t, 0)),
            pl.BlockSpec((1, _HALO, 1), lambda b, t: (b, (t + 1) * halo_idx, 0)),
            pl.BlockSpec((1, tile2, 1), lambda b, t: (b, t, 0)),
            pl.BlockSpec((16, C), lambda b, t: (0, 0)),
            pl.BlockSpec((1, C), lambda b, t: (0, 0)),
            pl.BlockSpec((3, 3, C, Co), lambda b, t: (0, 0, 0, 0)),
            pl.BlockSpec((1, Co), lambda b, t: (0, 0)),
            pl.BlockSpec((Fo2 * Co, D), lambda b, t: (0, 0)),
            pl.BlockSpec((1, D), lambda b, t: (0, 0)),
        ],
        out_specs=pl.BlockSpec((1, tile2, D), lambda b, t: (b, t, 0)),
        compiler_params=pltpu.CompilerParams(
            dimension_semantics=("parallel", "parallel"),
            vmem_limit_bytes=(64 * 1024 * 1024) * 3 // 4),
    )(patches, patches, pads1, pads1, pads2,
      w1p, b1r, w2r, b2r, wlr, blr)

    return out[:, :To2], out_pad2


# speedup vs baseline: 1.2583x; 1.1706x over previous
import numpy as np
import jax
import jax.numpy as jnp
from jax.experimental import pallas as pl
from jax.experimental.pallas import tpu as pltpu


_HALO = 8


def _fused_kernel(x_ref, xh_ref, pd1_ref, pd1h_ref, pd2_ref,
                  wb_ref, b1_ref, w2_ref, b2_ref, wl_ref, bl_ref,
                  out_ref):
    tile2 = out_ref.shape[1]
    f_in = x_ref.shape[2]
    co = w2_ref.shape[3]
    n1 = wb_ref.shape[1]
    fo1 = n1 // co
    fo2 = fo1 // 2
    rows1 = 2 * tile2

    xm = x_ref[0].reshape(rows1, 2, f_in)
    x0 = xm[:, 0]
    x1 = xm[:, 1]
    x2 = jnp.concatenate([x0[1:], xh_ref[0][:1]], axis=0)
    xcat = jnp.concatenate([x0, x1, x2], axis=1).astype(jnp.bfloat16)

    xh = xh_ref[0]
    xhcat = jnp.concatenate([xh[0:1], xh[1:2], xh[2:3]], axis=1
                            ).astype(jnp.bfloat16)

    def conv1(xc, pd, n):
        y = jnp.dot(xc, wb_ref[...], preferred_element_type=jnp.float32)
        y = jnp.maximum(y + b1_ref[...], 0.0)
        y = y * (1.0 - pd).reshape(n, 1)
        return y.astype(jnp.bfloat16).reshape(n, fo1, co)

    c1m = conv1(xcat, pd1_ref[0], rows1)
    c1h = conv1(xhcat, pd1h_ref[0][:1], 1)

    c1p = c1m.reshape(tile2, 2, fo1, co)
    r_even = c1p[:, 0]
    r_odd = c1p[:, 1]
    r_next = jnp.concatenate([r_even[1:], c1h], axis=0)

    acc = jnp.zeros((tile2 * fo2, co), jnp.float32)
    for kh, rset in enumerate((r_even, r_odd, r_next)):
        rp = rset.reshape(tile2, fo2, 2, co)
        e = rp[:, :, 0]
        o = rp[:, :, 1]
        e2 = jnp.concatenate([e[:, 1:], jnp.zeros_like(e[:, :1])], axis=1)
        for kw, tap in enumerate((e, o, e2)):
            acc = acc + jnp.dot(tap.reshape(tile2 * fo2, co), w2_ref[kh, kw],
                                preferred_element_type=jnp.float32)

    y = jnp.maximum(acc + b2_ref[...], 0.0)
    y = y.reshape(tile2, fo2, co) * (1.0 - pd2_ref[0]).reshape(tile2, 1, 1)
    a = y.astype(jnp.bfloat16).reshape(tile2, fo2 * co)
    out = jnp.dot(a, wl_ref[...], preferred_element_type=jnp.float32)
    out_ref[0] = out + bl_ref[...]


def _subsample_pad(p):
    t = p.shape[1]
    if t % 2:
        p = jnp.pad(p, ((0, 0), (0, 1)), constant_values=1.0)
    return p[:, ::2]


def _weight_spec(shape):
    ndim = len(shape)
    try:
        return pl.BlockSpec(shape, lambda b, t: (0,) * ndim,
                            pipeline_mode=pl.Buffered(1))
    except Exception:
        return pl.BlockSpec(shape, lambda b, t: (0,) * ndim)


def kernel(inputs, input_paddings, conv1_w, conv1_b, conv2_w, conv2_b,
           lin_w, lin_b, *, tile2=32):
    x = inputs.astype(jnp.float32)
    B, T, F = x.shape
    C = conv1_w.shape[-1]
    Co = conv2_w.shape[-1]
    D = lin_w.shape[-1]
    To1, Fo1 = (T + 1) // 2, (F + 1) // 2
    To2, Fo2 = (To1 + 1) // 2, (Fo1 + 1) // 2

    n_t = pl.cdiv(To2, tile2)
    Tp2 = n_t * tile2
    S1 = 2 * Tp2 + _HALO
    Tin = 4 * Tp2 + _HALO

    out_pad1 = _subsample_pad(input_paddings)
    out_pad2 = _subsample_pad(out_pad1)
    pads1 = jnp.pad(out_pad1, ((0, 0), (0, S1 - To1)),
                    constant_values=1.0)[..., None]
    pads2 = jnp.pad(out_pad2, ((0, 0), (0, Tp2 - To2)),
                    constant_values=1.0)[..., None]

    xp = jnp.pad(x, ((0, 0), (0, Tin - T), (0, 0)))

    wband = jnp.zeros((3, F, Fo1, C), jnp.float32)
    for dw in range(3):
        f_idx = np.arange(Fo1)
        k_idx = 2 * f_idx + dw
        keep = k_idx < F
        wband = wband.at[:, k_idx[keep], f_idx[keep], :].set(
            conv1_w[:, dw, 0, :][:, None, :])
    wband = wband.reshape(3 * F, Fo1 * C).astype(jnp.bfloat16)
    b1r = jnp.tile(conv1_b.reshape(1, 1, C), (1, Fo1, 1)
                   ).reshape(1, Fo1 * C).astype(jnp.float32)
    w2r = conv2_w.astype(jnp.bfloat16)
    b2r = conv2_b.reshape(1, Co).astype(jnp.float32)
    wlr = lin_w.astype(jnp.bfloat16)
    blr = lin_b.reshape(1, D).astype(jnp.float32)

    rows = 2 * tile2
    h_in = 4 * tile2 // _HALO
    h_c1 = rows // _HALO

    out = pl.pallas_call(
        _fused_kernel,
        out_shape=jax.ShapeDtypeStruct((B, Tp2, D), jnp.float32),
        grid=(B, n_t),
        in_specs=[
            pl.BlockSpec((1, 4 * tile2, F), lambda b, t: (b, t, 0)),
            pl.BlockSpec((1, _HALO, F), lambda b, t: (b, (t + 1) * h_in, 0)),
            pl.BlockSpec((1, rows, 1), lambda b, t: (b, t, 0)),
            pl.BlockSpec((1, _HALO, 1), lambda b, t: (b, (t + 1) * h_c1, 0)),
            pl.BlockSpec((1, tile2, 1), lambda b, t: (b, t, 0)),
            _weight_spec((3 * F, Fo1 * C)),
            _weight_spec((1, Fo1 * C)),
            _weight_spec((3, 3, C, Co)),
            _weight_spec((1, Co)),
            _weight_spec((Fo2 * Co, D)),
            _weight_spec((1, D)),
        ],
        out_specs=pl.BlockSpec((1, tile2, D), lambda b, t: (b, t, 0)),
        compiler_params=pltpu.CompilerParams(
            dimension_semantics=("parallel", "parallel"),
            vmem_limit_bytes=56 * 1024 * 1024),
    )(xp, xp, pads1, pads1, pads2,
      wband, b1r, w2r, b2r, wlr, blr)

    return out[:, :To2], out_pad2


# speedup vs baseline: 1.2651x; 1.0054x over previous
import numpy as np
import jax
import jax.numpy as jnp
from jax.experimental import pallas as pl
from jax.experimental.pallas import tpu as pltpu


_HALO = 8


def _fused_kernel(x_ref, xh_ref, pd1_ref, pd1h_ref, pd2_ref,
                  wb_ref, b1_ref, w2_ref, b2_ref, wl_ref, bl_ref,
                  out_ref):
    tile2 = out_ref.shape[1]
    f_in = x_ref.shape[2]
    co = w2_ref.shape[3]
    n1 = wb_ref.shape[1]
    fo1 = n1 // co
    fo2 = fo1 // 2
    rows1 = 2 * tile2

    xm = x_ref[0].reshape(rows1, 2, f_in)
    x0 = xm[:, 0]
    x1 = xm[:, 1]
    x2 = jnp.concatenate([x0[1:], xh_ref[0][:1]], axis=0)
    xcat = jnp.concatenate([x0, x1, x2], axis=1).astype(jnp.bfloat16)

    xh = xh_ref[0]
    xhcat = jnp.concatenate([xh[0:1], xh[1:2], xh[2:3]], axis=1
                            ).astype(jnp.bfloat16)

    def conv1(xc, pd, n):
        y = jnp.dot(xc, wb_ref[...], preferred_element_type=jnp.float32)
        y = jnp.maximum(y + b1_ref[...], 0.0)
        y = y * (1.0 - pd).reshape(n, 1)
        return y.astype(jnp.bfloat16).reshape(n, fo1, co)

    c1m = conv1(xcat, pd1_ref[0], rows1)
    c1h = conv1(xhcat, pd1h_ref[0][:1], 1)

    c1p = c1m.reshape(tile2, 2, fo1, co)
    r_even = c1p[:, 0]
    r_odd = c1p[:, 1]
    r_next = jnp.concatenate([r_even[1:], c1h], axis=0)

    acc = jnp.zeros((tile2 * fo2, co), jnp.float32)
    for kh, rset in enumerate((r_even, r_odd, r_next)):
        rp = rset.reshape(tile2, fo2, 2, co)
        e = rp[:, :, 0]
        o = rp[:, :, 1]
        e2 = jnp.concatenate([e[:, 1:], jnp.zeros_like(e[:, :1])], axis=1)
        for kw, tap in enumerate((e, o, e2)):
            acc = acc + jnp.dot(tap.reshape(tile2 * fo2, co), w2_ref[kh, kw],
                                preferred_element_type=jnp.float32)

    y = jnp.maximum(acc + b2_ref[...], 0.0)
    y = y.reshape(tile2, fo2, co) * (1.0 - pd2_ref[0]).reshape(tile2, 1, 1)
    a = y.astype(jnp.bfloat16).reshape(tile2, fo2 * co)
    out = jnp.dot(a, wl_ref[...], preferred_element_type=jnp.float32)
    out_ref[0] = out + bl_ref[...]


def _subsample_pad(p):
    t = p.shape[1]
    if t % 2:
        p = jnp.pad(p, ((0, 0), (0, 1)), constant_values=1.0)
    return p[:, ::2]


def _weight_spec(shape):
    ndim = len(shape)
    try:
        return pl.BlockSpec(shape, lambda b, t: (0,) * ndim,
                            pipeline_mode=pl.Buffered(1))
    except Exception:
        return pl.BlockSpec(shape, lambda b, t: (0,) * ndim)


def kernel(inputs, input_paddings, conv1_w, conv1_b, conv2_w, conv2_b,
           lin_w, lin_b, *, tile2=32):
    x = inputs.astype(jnp.float32)
    B, T, F = x.shape
    C = conv1_w.shape[-1]
    Co = conv2_w.shape[-1]
    D = lin_w.shape[-1]
    To1, Fo1 = (T + 1) // 2, (F + 1) // 2
    To2, Fo2 = (To1 + 1) // 2, (Fo1 + 1) // 2

    n_t = pl.cdiv(To2, tile2)
    Tp2 = n_t * tile2
    S1 = 2 * Tp2 + _HALO
    Tin = 4 * Tp2 + _HALO

    out_pad1 = _subsample_pad(input_paddings)
    out_pad2 = _subsample_pad(out_pad1)
    pads1 = jnp.pad(out_pad1, ((0, 0), (0, S1 - To1)),
                    constant_values=1.0)[..., None]
    pads2 = jnp.pad(out_pad2, ((0, 0), (0, Tp2 - To2)),
                    constant_values=1.0)[..., None]

    xp = jnp.pad(x, ((0, 0), (0, Tin - T), (0, 0)))

    sel = np.zeros((F, Fo1, 3), np.float32)
    for dw in range(3):
        for f in range(Fo1):
            if 2 * f + dw < F:
                sel[2 * f + dw, f, dw] = 1.0
    wband = jnp.einsum('kfw,dwc->dkfc', jnp.asarray(sel), conv1_w[:, :, 0, :])
    wband = wband.reshape(3 * F, Fo1 * C).astype(jnp.bfloat16)
    b1r = jnp.tile(conv1_b.reshape(1, 1, C), (1, Fo1, 1)
                   ).reshape(1, Fo1 * C).astype(jnp.float32)
    w2r = conv2_w.astype(jnp.bfloat16)
    b2r = conv2_b.reshape(1, Co).astype(jnp.float32)
    wlr = lin_w.astype(jnp.bfloat16)
    blr = lin_b.reshape(1, D).astype(jnp.float32)

    rows = 2 * tile2
    h_in = 4 * tile2 // _HALO
    h_c1 = rows // _HALO

    out = pl.pallas_call(
        _fused_kernel,
        out_shape=jax.ShapeDtypeStruct((B, Tp2, D), jnp.float32),
        grid=(B, n_t),
        in_specs=[
            pl.BlockSpec((1, 4 * tile2, F), lambda b, t: (b, t, 0)),
            pl.BlockSpec((1, _HALO, F), lambda b, t: (b, (t + 1) * h_in, 0)),
            pl.BlockSpec((1, rows, 1), lambda b, t: (b, t, 0)),
            pl.BlockSpec((1, _HALO, 1), lambda b, t: (b, (t + 1) * h_c1, 0)),
            pl.BlockSpec((1, tile2, 1), lambda b, t: (b, t, 0)),
            _weight_spec((3 * F, Fo1 * C)),
            _weight_spec((1, Fo1 * C)),
            _weight_spec((3, 3, C, Co)),
            _weight_spec((1, Co)),
            _weight_spec((Fo2 * Co, D)),
            _weight_spec((1, D)),
        ],
        out_specs=pl.BlockSpec((1, tile2, D), lambda b, t: (b, t, 0)),
        compiler_params=pltpu.CompilerParams(
            dimension_semantics=("parallel", "parallel"),
            vmem_limit_bytes=56 * 1024 * 1024),
    )(xp, xp, pads1, pads1, pads2,
      wband, b1r, w2r, b2r, wlr, blr)

    return out[:, :To2], out_pad2


# speedup vs baseline: 1.2791x; 1.0110x over previous
import numpy as np
import jax
import jax.numpy as jnp
from jax.experimental import pallas as pl
from jax.experimental.pallas import tpu as pltpu


_HALO = 8


def _fused_kernel(x_ref, xh_ref, pd1_ref, pd1h_ref, pd2_ref,
                  wb_ref, b1_ref, w2_ref, b2_ref, wl_ref, bl_ref,
                  out_ref):
    tile2 = out_ref.shape[1]
    f_in = x_ref.shape[2]
    co = w2_ref.shape[3]
    n1 = wb_ref.shape[1]
    fo1 = n1 // co
    fo2 = fo1 // 2
    rows1 = 2 * tile2

    xall = jnp.concatenate([x_ref[0], xh_ref[0]], axis=0)
    xr = xall.reshape((4 * tile2 + 8) // 2, 2, f_in)
    x0 = xr[:rows1 + 1, 0]
    x1 = xr[:rows1 + 1, 1]
    x2 = xr[1:rows1 + 2, 0]
    xcat = jnp.concatenate([x0, x1, x2], axis=1).astype(jnp.bfloat16)

    y = jnp.dot(xcat, wb_ref[...], preferred_element_type=jnp.float32)
    y = jnp.maximum(y + b1_ref[...], 0.0)
    pd1 = jnp.concatenate([pd1_ref[0], pd1h_ref[0]], axis=0)[:rows1 + 1]
    y = y * (1.0 - pd1).reshape(rows1 + 1, 1)
    c1 = y.astype(jnp.bfloat16).reshape(rows1 + 1, fo1, co)

    c1p = c1[:rows1].reshape(tile2, 2, fo1, co)
    r_even = c1p[:, 0]
    r_odd = c1p[:, 1]
    r_next = jnp.concatenate([r_even[1:], c1[rows1:rows1 + 1]], axis=0)

    acc = jnp.zeros((tile2 * fo2, co), jnp.float32)
    for kh, rset in enumerate((r_even, r_odd, r_next)):
        rp = rset.reshape(tile2, fo2, 2, co)
        e = rp[:, :, 0]
        o = rp[:, :, 1]
        e2 = jnp.concatenate([e[:, 1:], jnp.zeros_like(e[:, :1])], axis=1)
        for kw, tap in enumerate((e, o, e2)):
            acc = acc + jnp.dot(tap.reshape(tile2 * fo2, co), w2_ref[kh, kw],
                                preferred_element_type=jnp.float32)

    y = jnp.maximum(acc + b2_ref[...], 0.0)
    y = y.reshape(tile2, fo2, co) * (1.0 - pd2_ref[0]).reshape(tile2, 1, 1)
    a = y.astype(jnp.bfloat16).reshape(tile2, fo2 * co)
    out = jnp.dot(a, wl_ref[...], preferred_element_type=jnp.float32)
    out_ref[0] = out + bl_ref[...]


def _subsample_pad(p):
    t = p.shape[1]
    if t % 2:
        p = jnp.pad(p, ((0, 0), (0, 1)), constant_values=1.0)
    return p[:, ::2]


def _weight_spec(shape):
    ndim = len(shape)
    try:
        return pl.BlockSpec(shape, lambda b, t: (0,) * ndim,
                            pipeline_mode=pl.Buffered(1))
    except Exception:
        return pl.BlockSpec(shape, lambda b, t: (0,) * ndim)


def kernel(inputs, input_paddings, conv1_w, conv1_b, conv2_w, conv2_b,
           lin_w, lin_b, *, tile2=32):
    x = inputs.astype(jnp.float32)
    B, T, F = x.shape
    C = conv1_w.shape[-1]
    Co = conv2_w.shape[-1]
    D = lin_w.shape[-1]
    To1, Fo1 = (T + 1) // 2, (F + 1) // 2
    To2, Fo2 = (To1 + 1) // 2, (Fo1 + 1) // 2

    n_t = pl.cdiv(To2, tile2)
    Tp2 = n_t * tile2
    S1 = 2 * Tp2 + _HALO
    Tin = 4 * Tp2 + _HALO

    out_pad1 = _subsample_pad(input_paddings)
    out_pad2 = _subsample_pad(out_pad1)
    pads1 = jnp.pad(out_pad1, ((0, 0), (0, S1 - To1)),
                    constant_values=1.0)[..., None]
    pads2 = jnp.pad(out_pad2, ((0, 0), (0, Tp2 - To2)),
                    constant_values=1.0)[..., None]

    xp = jnp.pad(x, ((0, 0), (0, Tin - T), (0, 0)))

    sel = np.zeros((F, Fo1, 3), np.float32)
    for dw in range(3):
        for f in range(Fo1):
            if 2 * f + dw < F:
                sel[2 * f + dw, f, dw] = 1.0
    wband = jnp.einsum('kfw,dwc->dkfc', jnp.asarray(sel), conv1_w[:, :, 0, :])
    wband = wband.reshape(3 * F, Fo1 * C).astype(jnp.bfloat16)
    b1r = jnp.tile(conv1_b.reshape(1, 1, C), (1, Fo1, 1)
                   ).reshape(1, Fo1 * C).astype(jnp.float32)
    w2r = conv2_w.astype(jnp.bfloat16)
    b2r = conv2_b.reshape(1, Co).astype(jnp.float32)
    wlr = lin_w.astype(jnp.bfloat16)
    blr = lin_b.reshape(1, D).astype(jnp.float32)

    rows = 2 * tile2
    h_in = 4 * tile2 // _HALO
    h_c1 = rows // _HALO

    out = pl.pallas_call(
        _fused_kernel,
        out_shape=jax.ShapeDtypeStruct((B, Tp2, D), jnp.float32),
        grid=(B, n_t),
        in_specs=[
            pl.BlockSpec((1, 4 * tile2, F), lambda b, t: (b, t, 0)),
            pl.BlockSpec((1, _HALO, F), lambda b, t: (b, (t + 1) * h_in, 0)),
            pl.BlockSpec((1, rows, 1), lambda b, t: (b, t, 0)),
            pl.BlockSpec((1, _HALO, 1), lambda b, t: (b, (t + 1) * h_c1, 0)),
            pl.BlockSpec((1, tile2, 1), lambda b, t: (b, t, 0)),
            _weight_spec((3 * F, Fo1 * C)),
            _weight_spec((1, Fo1 * C)),
            _weight_spec((3, 3, C, Co)),
            _weight_spec((1, Co)),
            _weight_spec((Fo2 * Co, D)),
            _weight_spec((1, D)),
        ],
        out_specs=pl.BlockSpec((1, tile2, D), lambda b, t: (b, t, 0)),
        compiler_params=pltpu.CompilerParams(
            dimension_semantics=("parallel", "parallel"),
            vmem_limit_bytes=56 * 1024 * 1024),
    )(xp, xp, pads1, pads1, pads2,
      wband, b1r, w2r, b2r, wlr, blr)

    return out[:, :To2], out_pad2


# speedup vs baseline: 1.4697x; 1.1491x over previous
import numpy as np
import jax
import jax.numpy as jnp
from jax.experimental import pallas as pl
from jax.experimental.pallas import tpu as pltpu


_HALO = 8
_BBLK = 2


def _fused_kernel(x_ref, xh_ref, pd1_ref, pd1h_ref, pd2_ref,
                  wb_ref, b1_ref, w2_ref, b2_ref, wl_ref, bl_ref,
                  out_ref):
    bblk, tile2, d = out_ref.shape
    f_in = x_ref.shape[2]
    co = w2_ref.shape[3]
    n1 = wb_ref.shape[1]
    fo1 = n1 // co
    fo2 = fo1 // 2
    rows1 = 2 * tile2
    nr = rows1 + 1

    xcats, pd1s = [], []
    for b in range(bblk):
        xall = jnp.concatenate([x_ref[b], xh_ref[b]], axis=0)
        xr = xall.reshape((4 * tile2 + _HALO) // 2, 2, f_in)
        x0 = xr[:nr, 0]
        x1 = xr[:nr, 1]
        x2 = xr[1:nr + 1, 0]
        xcats.append(jnp.concatenate([x0, x1, x2], axis=1))
        pd1s.append(jnp.concatenate([pd1_ref[b], pd1h_ref[b]], axis=0)[:nr])
    xcat = jnp.concatenate(xcats, axis=0).astype(jnp.bfloat16)
    pd1 = jnp.concatenate(pd1s, axis=0)

    y = jnp.dot(xcat, wb_ref[...], preferred_element_type=jnp.float32)
    y = jnp.maximum(y + b1_ref[...], 0.0)
    y = y * (1.0 - pd1).reshape(bblk * nr, 1)
    c1 = y.astype(jnp.bfloat16).reshape(bblk, nr, fo1, co)

    evens, odds, nexts = [], [], []
    for b in range(bblk):
        c1p = c1[b, :rows1].reshape(tile2, 2, fo1, co)
        evens.append(c1p[:, 0])
        odds.append(c1p[:, 1])
        nexts.append(jnp.concatenate([c1p[1:, 0], c1[b, rows1:nr]], axis=0))
    r_even = jnp.concatenate(evens, axis=0)
    r_odd = jnp.concatenate(odds, axis=0)
    r_next = jnp.concatenate(nexts, axis=0)

    m2 = bblk * tile2 * fo2
    acc = jnp.zeros((m2, co), jnp.float32)
    for kh, rset in enumerate((r_even, r_odd, r_next)):
        rp = rset.reshape(bblk * tile2, fo2, 2, co)
        e = rp[:, :, 0]
        o = rp[:, :, 1]
        e2 = jnp.concatenate([e[:, 1:], jnp.zeros_like(e[:, :1])], axis=1)
        for kw, tap in enumerate((e, o, e2)):
            acc = acc + jnp.dot(tap.reshape(m2, co), w2_ref[kh, kw],
                                preferred_element_type=jnp.float32)

    y2 = jnp.maximum(acc + b2_ref[...], 0.0)
    y2 = (y2.reshape(bblk * tile2, fo2, co)
          * (1.0 - pd2_ref[...]).reshape(bblk * tile2, 1, 1))
    a = y2.astype(jnp.bfloat16).reshape(bblk * tile2, fo2 * co)
    out = jnp.dot(a, wl_ref[...], preferred_element_type=jnp.float32)
    out_ref[...] = (out + bl_ref[...]).reshape(bblk, tile2, d)


def _subsample_pad(p):
    t = p.shape[1]
    if t % 2:
        p = jnp.pad(p, ((0, 0), (0, 1)), constant_values=1.0)
    return p[:, ::2]


def _weight_spec(shape):
    ndim = len(shape)
    try:
        return pl.BlockSpec(shape, lambda b, t: (0,) * ndim,
                            pipeline_mode=pl.Buffered(1))
    except Exception:
        return pl.BlockSpec(shape, lambda b, t: (0,) * ndim)


def kernel(inputs, input_paddings, conv1_w, conv1_b, conv2_w, conv2_b,
           lin_w, lin_b, *, tile2=32):
    x = inputs.astype(jnp.float32)
    B, T, F = x.shape
    C = conv1_w.shape[-1]
    Co = conv2_w.shape[-1]
    D = lin_w.shape[-1]
    To1, Fo1 = (T + 1) // 2, (F + 1) // 2
    To2, Fo2 = (To1 + 1) // 2, (Fo1 + 1) // 2

    n_t = pl.cdiv(To2, tile2)
    Tp2 = n_t * tile2
    S1 = 2 * Tp2 + _HALO
    Tin = 4 * Tp2 + _HALO

    out_pad1 = _subsample_pad(input_paddings)
    out_pad2 = _subsample_pad(out_pad1)
    pads1 = jnp.pad(out_pad1, ((0, 0), (0, S1 - To1)),
                    constant_values=1.0)[..., None]
    pads2 = jnp.pad(out_pad2, ((0, 0), (0, Tp2 - To2)),
                    constant_values=1.0)[..., None]

    xp = jnp.pad(x, ((0, 0), (0, Tin - T), (0, 0)))

    sel = np.zeros((F, Fo1, 3), np.float32)
    for dw in range(3):
        for f in range(Fo1):
            if 2 * f + dw < F:
                sel[2 * f + dw, f, dw] = 1.0
    wband = jnp.einsum('kfw,dwc->dkfc', jnp.asarray(sel), conv1_w[:, :, 0, :])
    wband = wband.reshape(3 * F, Fo1 * C).astype(jnp.bfloat16)
    b1r = jnp.tile(conv1_b.reshape(1, 1, C), (1, Fo1, 1)
                   ).reshape(1, Fo1 * C).astype(jnp.float32)
    w2r = conv2_w.astype(jnp.bfloat16)
    b2r = conv2_b.reshape(1, Co).astype(jnp.float32)
    wlr = lin_w.astype(jnp.bfloat16)
    blr = lin_b.reshape(1, D).astype(jnp.float32)

    rows = 2 * tile2
    h_in = 4 * tile2 // _HALO
    h_c1 = rows // _HALO

    out = pl.pallas_call(
        _fused_kernel,
        out_shape=jax.ShapeDtypeStruct((B, Tp2, D), jnp.float32),
        grid=(B // _BBLK, n_t),
        in_specs=[
            pl.BlockSpec((_BBLK, 4 * tile2, F), lambda b, t: (b, t, 0)),
            pl.BlockSpec((_BBLK, _HALO, F),
                         lambda b, t: (b, (t + 1) * h_in, 0)),
            pl.BlockSpec((_BBLK, rows, 1), lambda b, t: (b, t, 0)),
            pl.BlockSpec((_BBLK, _HALO, 1),
                         lambda b, t: (b, (t + 1) * h_c1, 0)),
            pl.BlockSpec((_BBLK, tile2, 1), lambda b, t: (b, t, 0)),
            _weight_spec((3 * F, Fo1 * C)),
            _weight_spec((1, Fo1 * C)),
            _weight_spec((3, 3, C, Co)),
            _weight_spec((1, Co)),
            _weight_spec((Fo2 * Co, D)),
            _weight_spec((1, D)),
        ],
        out_specs=pl.BlockSpec((_BBLK, tile2, D), lambda b, t: (b, t, 0)),
        compiler_params=pltpu.CompilerParams(
            dimension_semantics=("parallel", "parallel"),
            vmem_limit_bytes=117 * 512 * 1024),
    )(xp, xp, pads1, pads1, pads2,
      wband, b1r, w2r, b2r, wlr, blr)

    return out[:, :To2], out_pad2
